# baseline (speedup 1.0000x reference)
import functools

import jax
import jax.numpy as jnp
from jax import lax
from jax.experimental import pallas as pl
from jax.experimental.pallas import tpu as pltpu
from jax.experimental.pallas import tpu_sc as plsc

N = 10000
E = 320000
H = 128

NP = 10240
NCORES = 1
NW = NCORES * 16
ROWS_PER_TILE = NP // 16
EDGES_PER_TILE = 327680 // NW
CHUNK = 128
NCHUNKS = EDGES_PER_TILE // CHUNK
EP = NW * EDGES_PER_TILE

_PREC = jax.lax.Precision.HIGHEST


def _dot(a, b):
    return jnp.dot(a, b, precision=_PREC, preferred_element_type=jnp.float32)


def _tc1_body(x_ref, w1_ref, b1_ref, w2_ref, b2_ref, w3_ref, b3_ref,
              w4_ref, b4_ref, wn_ref, h_ref, m_ref):
    h0 = jax.nn.relu(_dot(x_ref[...], w1_ref[...]) + b1_ref[...])
    h1 = jax.nn.relu(_dot(h0, w2_ref[...]) + b2_ref[...])
    h2 = jax.nn.relu(_dot(h1, w3_ref[...]) + b3_ref[...]) + h1
    h3 = jax.nn.relu(_dot(h2, w4_ref[...]) + b4_ref[...]) + h2
    h_ref[...] = h3
    m_ref[...] = _dot(h3, wn_ref[...])


def _tc2_body(h_ref, s_ref, degp_ref, ws_ref, b_ref, wn_ref,
              h1_ref, m2_ref, invd_ref):
    deg = jnp.sum(degp_ref[...], axis=0)
    invd = 1.0 / jnp.maximum(deg, 1.0)
    agg = jnp.sum(s_ref[...], axis=0) * invd[:, None]
    h1 = jax.nn.relu(_dot(h_ref[...], ws_ref[...]) + agg + b_ref[...])
    h1_ref[...] = h1
    m2_ref[...] = _dot(h1, wn_ref[...])
    invd_ref[...] = invd[:, None]


def _tc3_body(h1_ref, s_ref, invd_ref, ws_ref, b_ref,
              w5_ref, b5_ref, out_ref):
    agg = jnp.sum(s_ref[...], axis=0) * invd_ref[...]
    h2 = jax.nn.relu(_dot(h1_ref[...], ws_ref[...]) + agg + b_ref[...])
    out_ref[...] = _dot(h2, w5_ref[...]) + b5_ref[...]


BLK = 1280
GRID = NP // BLK


def _full(shape):
    return pl.BlockSpec(shape, lambda i: (0,) * len(shape))


def _rows(width):
    return pl.BlockSpec((BLK, width), lambda i: (i, 0))


def _tc1(xp, w1p, b1, w2, b2, w3, b3, w4, b4, wn):
    return pl.pallas_call(
        _tc1_body,
        grid=(GRID,),
        in_specs=[_rows(8), _full((8, 64)), _full((1, 64)), _full((64, H)),
                  _full((1, H)), _full((H, H)), _full((1, H)), _full((H, H)),
                  _full((1, H)), _full((H, H))],
        out_specs=[_rows(H), _rows(H)],
        out_shape=[jax.ShapeDtypeStruct((NP, H), jnp.float32),
                   jax.ShapeDtypeStruct((NP, H), jnp.float32)],
    )(xp, w1p, b1, w2, b2, w3, b3, w4, b4, wn)


def _seg_spec():
    return pl.BlockSpec((NCORES, BLK, H), lambda i: (0, i, 0))


def _tc2(h, seg, degp, ws, b, wn):
    return pl.pallas_call(
        _tc2_body,
        grid=(GRID,),
        in_specs=[_rows(H), _seg_spec(),
                  pl.BlockSpec((NW, BLK), lambda i: (0, i)),
                  _full((H, H)), _full((1, H)), _full((H, H))],
        out_specs=[_rows(H), _rows(H), _rows(1)],
        out_shape=[jax.ShapeDtypeStruct((NP, H), jnp.float32),
                   jax.ShapeDtypeStruct((NP, H), jnp.float32),
                   jax.ShapeDtypeStruct((NP, 1), jnp.float32)],
    )(h, seg, degp, ws, b, wn)


def _tc3(h1, seg, invd, ws, b, w5p, b5p):
    return pl.pallas_call(
        _tc3_body,
        grid=(GRID,),
        in_specs=[_rows(H), _seg_spec(), _rows(1),
                  _full((H, H)), _full((1, H)), _full((H, H)), _full((1, H))],
        out_specs=_rows(H),
        out_shape=jax.ShapeDtypeStruct((NP, H), jnp.float32),
    )(h1, seg, invd, ws, b, w5p, b5p)


QB = 16


def _seg_body(with_deg, *refs):
    if with_deg:
        (m_hbm, src_hbm, dst_hbm, z_hbm, seg_hbm, deg_hbm,
         src_q, dst_q, rows_v, degacc, sem0, sem1, acc) = refs
    else:
        (m_hbm, src_hbm, dst_hbm, z_hbm, seg_hbm,
         src_q, dst_q, rows_v, sem0, sem1, acc) = refs
    c = lax.axis_index("c")
    s = lax.axis_index("s")
    w = c * 16 + s

    base = s * ROWS_PER_TILE
    pltpu.sync_copy(z_hbm, acc.at[pl.ds(base, ROWS_PER_TILE)])
    if with_deg:
        zvec = jnp.zeros((16,), jnp.float32)
        ones = jnp.ones((16,), jnp.float32)

        def _zb(t, _):
            for k in range(8):
                degacc[t, pl.ds(k * 16, 16)] = zvec
            return 0
        lax.fori_loop(0, NP // H, _zb, 0)
    plsc.subcore_barrier()

    sems = (sem0, sem1)

    def _gather(jj, b):
        pltpu.async_copy(m_hbm.at[src_q.at[jj]], rows_v.at[b], sems[b])

    def _gwait(jj, b):
        pltpu.make_async_copy(m_hbm.at[src_q.at[jj]], rows_v.at[b],
                              sems[b]).wait()

    for q in range(NCHUNKS // QB):
        pltpu.sync_copy(src_hbm.at[w, pl.ds(q * QB, QB)], src_q)
        pltpu.sync_copy(dst_hbm.at[w, pl.ds(q * QB, QB)], dst_q)
        _gather(0, 0)
        _gather(1, 1)

        def _mb(i, _):
            for b in range(2):
                jj = i * 2 + b
                if with_deg:
                    for k in range(CHUNK // 16):
                        d = dst_q[jj, pl.ds(k * 16, 16)]
                        row = lax.shift_right_logical(d, 7)
                        col = lax.bitwise_and(d, 127)
                        plsc.addupdate_scatter(degacc, [row, col], ones)
                _gwait(jj, b)
                pltpu.sync_copy(rows_v.at[b], acc.at[dst_q.at[jj]], add=True)

                @pl.when(jj + 2 < QB)
                def _():
                    _gather(jj + 2, b)
            return 0
        lax.fori_loop(0, QB // 2, _mb, 0)

    plsc.subcore_barrier()

    pltpu.sync_copy(acc.at[pl.ds(base, ROWS_PER_TILE)],
                    seg_hbm.at[c, pl.ds(base, ROWS_PER_TILE)])
    if with_deg:
        pltpu.sync_copy(degacc, deg_hbm.at[w])


def _make_seg(with_deg):
    scratch = [
        pltpu.VMEM((QB, CHUNK), jnp.int32),
        pltpu.VMEM((QB, CHUNK), jnp.int32),
        pltpu.VMEM((2, CHUNK, H), jnp.float32),
    ]
    if with_deg:
        scratch.append(pltpu.VMEM((NP // H, H), jnp.float32))
    scratch += [
        pltpu.SemaphoreType.DMA,
        pltpu.SemaphoreType.DMA,
        pltpu.VMEM_SHARED((NP, H), jnp.float32),
    ]
    out_type = [jax.ShapeDtypeStruct((NCORES, NP, H), jnp.float32)]
    if with_deg:
        out_type.append(jax.ShapeDtypeStruct((NW, NP // H, H), jnp.float32))
    mesh = plsc.VectorSubcoreMesh(core_axis_name="c", subcore_axis_name="s",
                                  num_cores=NCORES)
    return pl.kernel(
        functools.partial(_seg_body, with_deg),
        out_type=out_type,
        mesh=mesh,
        scratch_types=scratch,
        compiler_params=pltpu.CompilerParams(needs_layout_passes=False),
    )


_seg_deg = _make_seg(True)
_seg_only = _make_seg(False)


def kernel(x, W1, b1, W2, b2, W3, b3, W4, b4,
           c1_ws, c1_wn, c1_b, c2_ws, c2_wn, c2_b,
           W5, b5, edge_index):
    xp = jnp.zeros((NP, 8), jnp.float32).at[:N, :5].set(x)
    w1p = jnp.zeros((8, 64), jnp.float32).at[:5].set(W1)
    w5p = jnp.zeros((H, H), jnp.float32).at[:, :60].set(W5)
    b5p = jnp.zeros((1, H), jnp.float32).at[0, :60].set(b5)

    pad = EP - E
    srcp = jnp.concatenate([edge_index[0],
                            jnp.zeros((pad,), jnp.int32)])
    dstp = jnp.concatenate([edge_index[1],
                            jnp.full((pad,), N, jnp.int32)])
    src3 = srcp.reshape(NW, NCHUNKS, CHUNK)
    dst3 = dstp.reshape(NW, NCHUNKS, CHUNK)
    zrs = jnp.zeros((ROWS_PER_TILE, H), jnp.float32)

    h, m1 = _tc1(xp, w1p, b1.reshape(1, 64), W2, b2.reshape(1, H),
                 W3, b3.reshape(1, H), W4, b4.reshape(1, H), c1_wn)

    seg1, degp = _seg_deg(m1, src3, dst3, zrs)
    degp = degp.reshape(NW, NP)
    h1, m2, invd = _tc2(h, seg1, degp,
                        c1_ws, c1_b.reshape(1, H), c2_wn)
    (seg2,) = _seg_only(m2, src3, dst3, zrs)
    outp = _tc3(h1, seg2, invd,
                c2_ws, c2_b.reshape(1, H), w5p, b5p)
    return outp[:N, :60]


# speedup vs baseline: 1.2577x; 1.2577x over previous
import functools

import jax
import jax.numpy as jnp
from jax import lax
from jax.experimental import pallas as pl
from jax.experimental.pallas import tpu as pltpu
from jax.experimental.pallas import tpu_sc as plsc

N = 10000
E = 320000
H = 128

NP = 10240
HW = H // 2
NCORES = 2
NW = NCORES * 16
ROWS_PER_TILE = NP // 16
EDGES_PER_TILE = 327680 // NW
CHUNK = 128
NCHUNKS = EDGES_PER_TILE // CHUNK
EP = NW * EDGES_PER_TILE

_PREC = jax.lax.Precision.DEFAULT


def _dot(a, b):
    return jnp.dot(a, b, precision=_PREC, preferred_element_type=jnp.float32)


def _tc1_body(x_ref, w1_ref, b1_ref, w2_ref, b2_ref, w3_ref, b3_ref,
              w4_ref, b4_ref, wn_ref, h_ref, m_ref):
    h0 = jax.nn.relu(_dot(x_ref[...], w1_ref[...]) + b1_ref[...])
    h1 = jax.nn.relu(_dot(h0, w2_ref[...]) + b2_ref[...])
    h2 = jax.nn.relu(_dot(h1, w3_ref[...]) + b3_ref[...]) + h1
    h3 = jax.nn.relu(_dot(h2, w4_ref[...]) + b4_ref[...]) + h2
    h_ref[...] = h3
    m_ref[...] = _dot(h3, wn_ref[...])


def _tc2_body(h_ref, s_ref, degp_ref, ws_ref, b_ref, wn_ref,
              h1_ref, m2_ref, invd_ref):
    deg = jnp.sum(degp_ref[...], axis=0)
    invd = 1.0 / jnp.maximum(deg, 1.0)
    agg = jnp.sum(s_ref[...], axis=0) * invd[:, None]
    h1 = jax.nn.relu(_dot(h_ref[...], ws_ref[...]) + agg + b_ref[...])
    h1_ref[...] = h1
    m2_ref[...] = _dot(h1, wn_ref[...])
    invd_ref[...] = invd[:, None]


def _tc3_body(h1_ref, s_ref, invd_ref, ws_ref, b_ref,
              w5_ref, b5_ref, out_ref):
    agg = jnp.sum(s_ref[...], axis=0) * invd_ref[...]
    h2 = jax.nn.relu(_dot(h1_ref[...], ws_ref[...]) + agg + b_ref[...])
    out_ref[...] = _dot(h2, w5_ref[...]) + b5_ref[...]


BLK = 1280
GRID = NP // BLK


def _full(shape):
    return pl.BlockSpec(shape, lambda i: (0,) * len(shape))


def _rows(width):
    return pl.BlockSpec((BLK, width), lambda i: (i, 0))


def _tc1(xp, w1p, b1, w2, b2, w3, b3, w4, b4, wn):
    return pl.pallas_call(
        _tc1_body,
        grid=(GRID,),
        in_specs=[_rows(8), _full((8, 64)), _full((1, 64)), _full((64, H)),
                  _full((1, H)), _full((H, H)), _full((1, H)), _full((H, H)),
                  _full((1, H)), _full((H, H))],
        out_specs=[_rows(H), _rows(H)],
        out_shape=[jax.ShapeDtypeStruct((NP, H), jnp.float32),
                   jax.ShapeDtypeStruct((NP, H), jnp.float32)],
    )(xp, w1p, b1, w2, b2, w3, b3, w4, b4, wn)


def _seg_spec():
    return pl.BlockSpec((NCORES, BLK, H), lambda i: (0, i, 0))


def _tc2(h, seg, degp, ws, b, wn):
    return pl.pallas_call(
        _tc2_body,
        grid=(GRID,),
        in_specs=[_rows(H), _seg_spec(),
                  pl.BlockSpec((NW, BLK), lambda i: (0, i)),
                  _full((H, H)), _full((1, H)), _full((H, H))],
        out_specs=[_rows(H), _rows(H), _rows(1)],
        out_shape=[jax.ShapeDtypeStruct((NP, H), jnp.float32),
                   jax.ShapeDtypeStruct((NP, H), jnp.float32),
                   jax.ShapeDtypeStruct((NP, 1), jnp.float32)],
    )(h, seg, degp, ws, b, wn)


def _tc3(h1, seg, invd, ws, b, w5p, b5p):
    return pl.pallas_call(
        _tc3_body,
        grid=(GRID,),
        in_specs=[_rows(H), _seg_spec(), _rows(1),
                  _full((H, H)), _full((1, H)), _full((H, H)), _full((1, H))],
        out_specs=_rows(H),
        out_shape=jax.ShapeDtypeStruct((NP, H), jnp.float32),
    )(h1, seg, invd, ws, b, w5p, b5p)


def _seg_body(with_deg, *refs):
    if with_deg:
        (m_hbm, src_hbm, dst_hbm, seg_hbm, deg_hbm,
         src_v, dst_v, rows_v, zeros_v, degacc, sem0, sem1, acc) = refs
    else:
        (m_hbm, src_hbm, dst_hbm, seg_hbm,
         src_v, dst_v, rows_v, zeros_v, sem0, sem1, acc) = refs
    c = lax.axis_index("c")
    s = lax.axis_index("s")
    w = c * 16 + s
    base = s * ROWS_PER_TILE

    zvec = jnp.zeros((16,), jnp.float32)
    for r in range(32):
        for k in range(8):
            zeros_v[r, pl.ds(k * 16, 16)] = zvec
    for i in range(ROWS_PER_TILE // 32):
        pltpu.sync_copy(zeros_v, acc.at[pl.ds(base + i * 32, 32)])
    if with_deg:
        ones = jnp.ones((16,), jnp.float32)

        def _zb(t, _):
            for k in range(8):
                degacc[t, pl.ds(k * 16, 16)] = zvec
            return 0
        lax.fori_loop(0, NP // H, _zb, 0)
    plsc.subcore_barrier()

    sems = (sem0, sem1)

    def _load_idx(j, b):
        pltpu.sync_copy(src_hbm.at[w, j], src_v.at[b])
        pltpu.sync_copy(dst_hbm.at[w, j], dst_v.at[b])

    def _gather(b):
        pltpu.async_copy(m_hbm.at[src_v.at[b]], rows_v.at[b], sems[b])

    def _gwait(b):
        pltpu.make_async_copy(m_hbm.at[src_v.at[b]], rows_v.at[b],
                              sems[b]).wait()

    _load_idx(0, 0)
    _gather(0)
    _load_idx(1, 1)
    _gather(1)

    def _mb(i, _):
        for b in range(2):
            jj = i * 2 + b
            if with_deg:
                for k in range(CHUNK // 16):
                    d = dst_v[b, pl.ds(k * 16, 16)]
                    row = lax.shift_right_logical(d, 7)
                    col = lax.bitwise_and(d, 127)
                    plsc.addupdate_scatter(degacc, [row, col], ones)
            _gwait(b)
            pltpu.sync_copy(rows_v.at[b], acc.at[dst_v.at[b]], add=True)

            @pl.when(jj + 2 < NCHUNKS)
            def _():
                _load_idx(jj + 2, b)
                _gather(b)
        return 0
    lax.fori_loop(0, NCHUNKS // 2, _mb, 0)

    plsc.subcore_barrier()
    pltpu.sync_copy(acc.at[pl.ds(base, ROWS_PER_TILE)],
                    seg_hbm.at[c, pl.ds(base, ROWS_PER_TILE)])
    if with_deg:
        pltpu.sync_copy(degacc, deg_hbm.at[w])


def _make_seg(with_deg):
    scratch = [
        pltpu.VMEM((2, CHUNK), jnp.int32),
        pltpu.VMEM((2, CHUNK), jnp.int32),
        pltpu.VMEM((2, CHUNK, H), jnp.float32),
        pltpu.VMEM((32, H), jnp.float32),
    ]
    if with_deg:
        scratch.append(pltpu.VMEM((NP // H, H), jnp.float32))
    scratch += [
        pltpu.SemaphoreType.DMA,
        pltpu.SemaphoreType.DMA,
        pltpu.VMEM_SHARED((NP, H), jnp.float32),
    ]
    out_type = [jax.ShapeDtypeStruct((NCORES, NP, H), jnp.float32)]
    if with_deg:
        out_type.append(jax.ShapeDtypeStruct((NW, NP // H, H), jnp.float32))
    mesh = plsc.VectorSubcoreMesh(core_axis_name="c", subcore_axis_name="s",
                                  num_cores=NCORES)
    return pl.kernel(
        functools.partial(_seg_body, with_deg),
        out_type=out_type,
        mesh=mesh,
        scratch_types=scratch,
        compiler_params=pltpu.CompilerParams(needs_layout_passes=False),
    )


_seg_deg = _make_seg(True)
_seg_only = _make_seg(False)


def kernel(x, W1, b1, W2, b2, W3, b3, W4, b4,
           c1_ws, c1_wn, c1_b, c2_ws, c2_wn, c2_b,
           W5, b5, edge_index):
    xp = jnp.zeros((NP, 8), jnp.float32).at[:N, :5].set(x)
    w1p = jnp.zeros((8, 64), jnp.float32).at[:5].set(W1)
    w5p = jnp.zeros((H, H), jnp.float32).at[:, :60].set(W5)
    b5p = jnp.zeros((1, H), jnp.float32).at[0, :60].set(b5)

    pad = EP - E
    srcp = jnp.concatenate([edge_index[0],
                            jnp.zeros((pad,), jnp.int32)])
    dstp = jnp.concatenate([edge_index[1],
                            jnp.full((pad,), N, jnp.int32)])
    src3 = srcp.reshape(NW, NCHUNKS, CHUNK)
    dst3 = dstp.reshape(NW, NCHUNKS, CHUNK)

    h, m1 = _tc1(xp, w1p, b1.reshape(1, 64), W2, b2.reshape(1, H),
                 W3, b3.reshape(1, H), W4, b4.reshape(1, H), c1_wn)

    seg1, degp = _seg_deg(m1, src3, dst3)
    degp = degp.reshape(NW, NP)
    h1, m2, invd = _tc2(h, seg1, degp,
                        c1_ws, c1_b.reshape(1, H), c2_wn)
    (seg2,) = _seg_only(m2, src3, dst3)
    outp = _tc3(h1, seg2, invd,
                c2_ws, c2_b.reshape(1, H), w5p, b5p)
    return outp[:N, :60]
